# baseline (speedup 1.0000x reference)
import functools

import numpy as np
import jax
import jax.numpy as jnp
from jax import lax
from jax.experimental import pallas as pl
from jax.experimental.pallas import tpu as pltpu

_VOCAB = 1000000
_D = 64
_B = 1024
_S = 200
_T = _B * _S
_NELEM = _T * _D
_SCALE = np.float32(5.0 / np.sqrt(_S * _D))

_LANES = 128
_NROWS = _NELEM // _LANES
_BLK = 512
_GRID = _NROWS // _BLK


def _threefry_eps(p):
    ks0 = jnp.uint32(0)
    ks1 = jnp.uint32(1)
    ks2 = jnp.uint32(0x1BD11BDB)
    x0 = jnp.full_like(p, ks0)
    x1 = p + ks1
    rot0 = (13, 15, 26, 6)
    rot1 = (17, 29, 16, 24)
    schedule = (
        (rot0, ks1, ks2, 1),
        (rot1, ks2, ks0, 2),
        (rot0, ks0, ks1, 3),
        (rot1, ks1, ks2, 4),
        (rot0, ks2, ks0, 5),
    )
    for rots, ka, kb, c in schedule:
        for r in rots:
            x0 = x0 + x1
            x1 = (x1 << jnp.uint32(r)) | (x1 >> jnp.uint32(32 - r))
            x1 = x0 ^ x1
        x0 = x0 + ka
        x1 = x1 + kb + jnp.uint32(c)
    bits = x0 ^ x1
    fbits = (bits >> jnp.uint32(9)) | jnp.uint32(0x3F800000)
    return lax.bitcast_convert_type(fbits, jnp.float32) - jnp.float32(1.0)


def _noise_add_body(x_ref, o_ref):
    b = pl.program_id(0)
    base = b.astype(jnp.uint32) * jnp.uint32(_BLK * _LANES)
    i = lax.broadcasted_iota(jnp.uint32, (_BLK, _LANES), 0)
    j = lax.broadcasted_iota(jnp.uint32, (_BLK, _LANES), 1)
    p = base + i * jnp.uint32(_LANES) + j
    o_ref[...] = x_ref[...] + _SCALE * _threefry_eps(p)


def _noise_add(xs2d, interpret=False):
    return pl.pallas_call(
        _noise_add_body,
        grid=(_GRID,),
        in_specs=[pl.BlockSpec((_BLK, _LANES), lambda b: (b, 0))],
        out_specs=pl.BlockSpec((_BLK, _LANES), lambda b: (b, 0)),
        out_shape=jax.ShapeDtypeStruct((_NROWS, _LANES), jnp.float32),
        interpret=interpret,
    )(xs2d)


def kernel(input_ids, table):
    xs = jnp.take(table, input_ids.reshape(-1), axis=0)
    out2d = _noise_add(xs.reshape(_NROWS, _LANES))
    return out2d.reshape(_B, _S, _D)
